# baseline (speedup 1.0000x reference)
import jax
import jax.numpy as jnp
from jax import lax
from jax.experimental import pallas as pl
from jax.experimental.pallas import tpu as pltpu
from jax.experimental.pallas import tpu_sc as plsc

_B = 16384
_NS = 26
_ND = 13
_NF = _NS + _ND
_V = 100000
_L = 16
_NC = 2
_NSUB = 16
_NW = _NC * _NSUB
_RPT = _B // _NW
_NG = _RPT // _L
_SPLIT = (15, 11)


def _make_body(f0, fc, first):
    def body(*refs):
        xt_hbm = refs[0]
        tabs = refs[1:1 + fc]
        aux_hbm = refs[1 + fc]
        out_hbm = refs[2 + fc]
        if first:
            xv, dv, idxv, gv, accv, wv = refs[3 + fc:3 + fc + 6]
            sems = refs[3 + fc + 6:]
        else:
            xv, idxv, gv, accv = refs[3 + fc:3 + fc + 4]
            sems = refs[3 + fc + 4:]

        wid = lax.axis_index("s") * _NC + lax.axis_index("c")
        base = wid * _RPT
        pltpu.sync_copy(xt_hbm.at[pl.ds(f0, fc), pl.ds(base, _RPT)], xv)
        if first:
            pltpu.sync_copy(xt_hbm.at[pl.ds(_NS, _ND), pl.ds(base, _RPT)], dv)
            pltpu.sync_copy(aux_hbm, wv)
        else:
            pltpu.sync_copy(aux_hbm.at[pl.ds(base, _RPT)], accv)

        handles = []
        for f in range(fc):
            def build(i, carry, f=f):
                off = i * _L
                idxv[f, pl.ds(off, _L)] = (
                    xv[f, pl.ds(off, _L)].astype(jnp.int32))
                return carry

            lax.fori_loop(0, _NG, build, 0)
            handles.append(
                pltpu.async_copy(tabs[f].at[idxv.at[f]], gv.at[f], sems[f]))

        if first:
            def dense(i, carry):
                off = i * _L
                acc = jnp.zeros((_L,), jnp.float32)
                for d in range(_ND):
                    acc = acc + dv[d, pl.ds(off, _L)] * wv[d]
                accv[pl.ds(off, _L)] = acc
                return carry

            lax.fori_loop(0, _NG, dense, 0)

        for f in range(fc):
            handles[f].wait()

            def accum(i, carry, f=f):
                off = i * _L
                accv[pl.ds(off, _L)] = (
                    accv[pl.ds(off, _L)] + gv[f, pl.ds(off, _L)])
                return carry

            lax.fori_loop(0, _NG, accum, 0)

        pltpu.sync_copy(accv, out_hbm.at[pl.ds(base, _RPT)])

    return body


def kernel(X, emb_tables, dense_weight):
    xt = X.T
    w16 = jnp.broadcast_to(dense_weight.reshape(_ND, 1), (_ND, _L))
    tabs = tuple(emb_tables[f, :, 0] for f in range(_NS))
    mesh = plsc.VectorSubcoreMesh(core_axis_name="c", subcore_axis_name="s")
    params = pltpu.CompilerParams(
        needs_layout_passes=False, use_tc_tiling_on_sc=False
    )

    part = w16
    f0 = 0
    for ci, fc in enumerate(_SPLIT):
        first = ci == 0
        scratch = [pltpu.VMEM((fc, _RPT), jnp.float32)]
        if first:
            scratch.append(pltpu.VMEM((_ND, _RPT), jnp.float32))
        scratch += [
            pltpu.VMEM((fc, _RPT), jnp.int32),
            pltpu.VMEM((fc, _RPT), jnp.float32),
            pltpu.VMEM((_RPT,), jnp.float32),
        ]
        if first:
            scratch.append(pltpu.VMEM((_ND, _L), jnp.float32))
        scratch.extend([pltpu.SemaphoreType.DMA] * fc)
        part = pl.kernel(
            _make_body(f0, fc, first),
            out_type=jax.ShapeDtypeStruct((_B,), jnp.float32),
            mesh=mesh,
            compiler_params=params,
            scratch_types=scratch,
        )(xt, *tabs[f0:f0 + fc], part)
        f0 += fc
    return part.reshape(_B, 1)


# speedup vs baseline: 1.0070x; 1.0070x over previous
import jax
import jax.numpy as jnp
from jax import lax
from jax.experimental import pallas as pl
from jax.experimental.pallas import tpu as pltpu
from jax.experimental.pallas import tpu_sc as plsc

_B = 16384
_NS = 26
_ND = 13
_NF = _NS + _ND
_V = 100000
_L = 16
_NC = 2
_NSUB = 16
_NW = _NC * _NSUB
_RPT = _B // _NW
_NG = _RPT // _L
_SPLIT = (15, 11)


def _make_body(f0, fc, first):
    def body(*refs):
        xt_hbm = refs[0]
        tabs = refs[1:1 + fc]
        aux_hbm = refs[1 + fc]
        out_hbm = refs[2 + fc]
        if first:
            xv, dv, idxv, gv, accv, wv = refs[3 + fc:3 + fc + 6]
            sems = refs[3 + fc + 6:]
        else:
            xv, idxv, gv, accv = refs[3 + fc:3 + fc + 4]
            sems = refs[3 + fc + 4:]

        wid = lax.axis_index("s") * _NC + lax.axis_index("c")
        base = wid * _RPT
        pltpu.sync_copy(xt_hbm.at[pl.ds(f0, fc), pl.ds(base, _RPT)], xv)
        if first:
            pltpu.sync_copy(xt_hbm.at[pl.ds(_NS, _ND), pl.ds(base, _RPT)], dv)
            pltpu.sync_copy(aux_hbm, wv)
        else:
            pltpu.sync_copy(aux_hbm.at[pl.ds(base, _RPT)], accv)

        handles = []
        for f in range(fc):
            def build(i, carry, f=f):
                off = i * _L
                idxv[f, pl.ds(off, _L)] = (
                    xv[f, pl.ds(off, _L)].astype(jnp.int32))
                return carry

            lax.fori_loop(0, _NG, build, 0)
            handles.append(
                pltpu.async_copy(tabs[f].at[idxv.at[f]], gv.at[f], sems[f]))

        if first:
            def dense(i, carry):
                off = i * _L
                acc = jnp.zeros((_L,), jnp.float32)
                for d in range(_ND):
                    acc = acc + dv[d, pl.ds(off, _L)] * wv[d]
                accv[pl.ds(off, _L)] = acc
                return carry

            lax.fori_loop(0, _NG, dense, 0)

        for g0 in range(0, fc, 4):
            gn = min(4, fc - g0)
            for f in range(g0, g0 + gn):
                handles[f].wait()

            def accum(i, carry, g0=g0, gn=gn):
                off = i * _L
                acc = accv[pl.ds(off, _L)]
                for f in range(g0, g0 + gn):
                    acc = acc + gv[f, pl.ds(off, _L)]
                accv[pl.ds(off, _L)] = acc
                return carry

            lax.fori_loop(0, _NG, accum, 0)

        pltpu.sync_copy(accv, out_hbm.at[pl.ds(base, _RPT)])

    return body


def kernel(X, emb_tables, dense_weight):
    xt = X.T
    w16 = jnp.broadcast_to(dense_weight.reshape(_ND, 1), (_ND, _L))
    tabs = tuple(emb_tables[f, :, 0] for f in range(_NS))
    mesh = plsc.VectorSubcoreMesh(core_axis_name="c", subcore_axis_name="s")
    params = pltpu.CompilerParams(
        needs_layout_passes=False, use_tc_tiling_on_sc=False
    )

    part = w16
    f0 = 0
    for ci, fc in enumerate(_SPLIT):
        first = ci == 0
        scratch = [pltpu.VMEM((fc, _RPT), jnp.float32)]
        if first:
            scratch.append(pltpu.VMEM((_ND, _RPT), jnp.float32))
        scratch += [
            pltpu.VMEM((fc, _RPT), jnp.int32),
            pltpu.VMEM((fc, _RPT), jnp.float32),
            pltpu.VMEM((_RPT,), jnp.float32),
        ]
        if first:
            scratch.append(pltpu.VMEM((_ND, _L), jnp.float32))
        scratch.extend([pltpu.SemaphoreType.DMA] * fc)
        part = pl.kernel(
            _make_body(f0, fc, first),
            out_type=jax.ShapeDtypeStruct((_B,), jnp.float32),
            mesh=mesh,
            compiler_params=params,
            scratch_types=scratch,
        )(xt, *tabs[f0:f0 + fc], part)
        f0 += fc
    return part.reshape(_B, 1)
